# baseline (speedup 1.0000x reference)
import functools

import jax
import jax.numpy as jnp
from jax import lax
from jax.experimental import pallas as pl
from jax.experimental.pallas import tpu as pltpu
from jax.experimental.pallas import tpu_sc as plsc

_N = 10000
_E = 320000
_D = 128
_H = _D // 2

_NC = 2
_NS = 16
_C = 128
_TOTCH = 2560
_CPT = _TOTCH // _NS
_EPAD = _TOTCH * _C
_NP = 10240
_ROWS_PT = _NP // _NS
_ZR = 128
_NB = 8

_mesh = plsc.VectorSubcoreMesh(core_axis_name="c", subcore_axis_name="s")


@functools.partial(
    pl.kernel,
    out_type=jax.ShapeDtypeStruct((_NC, _NP, _H), jnp.float32),
    mesh=_mesh,
    compiler_params=pltpu.CompilerParams(use_tc_tiling_on_sc=False),
    scratch_types=[
        pltpu.VMEM((_NB, 2, _C), jnp.int32),
        pltpu.VMEM((_NB, _C), jnp.float32),
        pltpu.VMEM((_NB, _C, _H), jnp.float32),
        pltpu.VMEM_SHARED((_NP, _H), jnp.float32),
        [pltpu.SemaphoreType.DMA] * _NB,
        [pltpu.SemaphoreType.DMA] * _NB,
    ],
)
def _sc_gather_scatter(h_hbm, edata_hbm, kv_hbm, out_hbm,
                       ed_v, kv_v, rows_v, agg_sh, esems, gsems):
    cid = lax.axis_index("c")
    sid = lax.axis_index("s")
    base = sid * _CPT

    def zbody(r, carry):
        for j in range(_H // 16):
            rows_v[0, r, pl.ds(j * 16, 16)] = jnp.zeros((16,), jnp.float32)
        return carry
    lax.fori_loop(0, _ZR, zbody, 0)
    row0 = sid * _ROWS_PT
    for j in range(_ROWS_PT // _ZR):
        pltpu.sync_copy(rows_v.at[0],
                        agg_sh.at[pl.ds(row0 + j * _ZR, _ZR)])
    plsc.subcore_barrier()

    def start_edata(k, b):
        kc = base + jnp.where(k < _CPT, k, 0)
        pltpu.async_copy(edata_hbm.at[kc], ed_v.at[b], esems[b])
        pltpu.async_copy(kv_hbm.at[kc], kv_v.at[b], esems[b])

    def wait_edata(b):
        pltpu.make_async_copy(edata_hbm.at[0], ed_v.at[b],
                              esems[b]).wait()
        pltpu.make_async_copy(kv_hbm.at[0], kv_v.at[b],
                              esems[b]).wait()

    def start_gather(b):
        pltpu.async_copy(h_hbm.at[cid].at[ed_v.at[b, 0]], rows_v.at[b],
                         gsems[b])

    def wait_gather(b):
        pltpu.make_async_copy(h_hbm.at[0, pl.ds(0, _C)], rows_v.at[b],
                              gsems[b]).wait()

    def scale_chunk(b):
        def gbody(g, carry):
            kv16 = kv_v[b, pl.ds(g * 16, 16)]
            for e16 in range(16):
                kv_e = kv16[e16]
                row = g * 16 + e16
                for j in range(_H // 16):
                    sl = pl.ds(j * 16, 16)
                    rows_v[b, row, sl] = rows_v[b, row, sl] * kv_e
            return carry
        lax.fori_loop(0, _C // 16, gbody, 0)

    for b in range(_NB):
        start_edata(b, b)
    for b in range(_NB - 1):
        wait_edata(b)
        start_gather(b)

    def outer(i, carry):
        for u in range(_NB):
            k = _NB * i + u
            b = u
            b3 = (u + _NB - 1) % _NB
            wait_edata(b3)
            start_gather(b3)
            wait_gather(b)
            start_edata(k + _NB, b)
        return carry
    lax.fori_loop(0, _CPT // _NB, outer, 0)
    for b in range(_NB - 1):
        wait_gather(b)
    wait_edata(_NB - 1)

    plsc.subcore_barrier()
    pltpu.sync_copy(agg_sh.at[pl.ds(row0, _ROWS_PT)],
                    out_hbm.at[cid, pl.ds(row0, _ROWS_PT)])


_R = 2048


def _mm_tanh_body(q_ref, w_ref, o_ref, osp_ref):
    w = w_ref[...]
    a = jnp.dot(q_ref[0], w[:_H], preferred_element_type=jnp.float32)
    a += jnp.dot(q_ref[1], w[_H:], preferred_element_type=jnp.float32)
    h = jnp.tanh(a)
    o_ref[...] = h
    osp_ref[0] = h[:, :_H]
    osp_ref[1] = h[:, _H:]


def _tc_mm_tanh(q, w):
    return pl.pallas_call(
        _mm_tanh_body,
        grid=(_NP // _R,),
        in_specs=[
            pl.BlockSpec((2, _R, _H), lambda i: (0, i, 0)),
            pl.BlockSpec((_D, _D), lambda i: (0, 0)),
        ],
        out_specs=[
            pl.BlockSpec((_R, _D), lambda i: (i, 0)),
            pl.BlockSpec((2, _R, _H), lambda i: (0, i, 0)),
        ],
        out_shape=[
            jax.ShapeDtypeStruct((_NP, _D), jnp.float32),
            jax.ShapeDtypeStruct((2, _NP, _H), jnp.float32),
        ],
    )(q, w)


def _final_body(x_ref, h1_ref, q_ref, w1_ref, we_ref, b_ref, o_ref):
    w1 = w1_ref[...]
    a = jnp.dot(q_ref[0], w1[:_H], preferred_element_type=jnp.float32)
    a += jnp.dot(q_ref[1], w1[_H:], preferred_element_type=jnp.float32)
    h2 = jnp.tanh(a)
    we = we_ref[...]
    acc = jnp.dot(x_ref[...], we[0:_D], preferred_element_type=jnp.float32)
    acc += jnp.dot(h1_ref[...], we[_D:2 * _D],
                   preferred_element_type=jnp.float32)
    acc += jnp.dot(h2, we[2 * _D:3 * _D], preferred_element_type=jnp.float32)
    o_ref[...] = acc + b_ref[...]


def _tc_final(x, h1, q, w1, w_enc, b_enc):
    return pl.pallas_call(
        _final_body,
        grid=(_NP // _R,),
        in_specs=[
            pl.BlockSpec((_R, _D), lambda i: (i, 0)),
            pl.BlockSpec((_R, _D), lambda i: (i, 0)),
            pl.BlockSpec((2, _R, _H), lambda i: (0, i, 0)),
            pl.BlockSpec((_D, _D), lambda i: (0, 0)),
            pl.BlockSpec((3 * _D, _D), lambda i: (0, 0)),
            pl.BlockSpec((1, _D), lambda i: (0, 0)),
        ],
        out_specs=pl.BlockSpec((_R, _D), lambda i: (i, 0)),
        out_shape=jax.ShapeDtypeStruct((_NP, _D), jnp.float32),
    )(x, h1, q, w1, w_enc, b_enc)


@jax.jit
def kernel(x, edge_index, kernel_vals, W0, W1, W_enc, b_enc):
    pad = _EPAD - _E
    src = jnp.concatenate([edge_index[0], jnp.zeros((pad,), jnp.int32)])
    dst = jnp.concatenate([edge_index[1], jnp.zeros((pad,), jnp.int32)])
    kv = jnp.concatenate([kernel_vals, jnp.zeros((pad,), jnp.float32)])
    edata = jnp.stack(
        [a.reshape(_TOTCH, _C) for a in (src, dst)], axis=1)
    kv_r = kv.reshape(_TOTCH, _C)
    x_p = jnp.pad(x, ((0, _NP - _N), (0, 0)))
    x_sp = jnp.stack([x_p[:, :_H], x_p[:, _H:]])

    p = _sc_gather_scatter(x_sp, edata, kv_r)
    h1, h1_sp = _tc_mm_tanh(p, W0)
    q = _sc_gather_scatter(h1_sp, edata, kv_r)
    emb = _tc_final(x_p, h1, q, W1, W_enc, b_enc.reshape(1, _D))
    return emb[:_N]


# speedup vs baseline: 1.2217x; 1.2217x over previous
import functools

import jax
import jax.numpy as jnp
from jax import lax
from jax.experimental import pallas as pl
from jax.experimental.pallas import tpu as pltpu
from jax.experimental.pallas import tpu_sc as plsc

_N = 10000
_E = 320000
_D = 128

_NC = 2
_NS = 16
_NW = _NC * _NS
_C = 120
_CPT = 84
_TOTCH = _NW * _CPT
_EPAD = _TOTCH * _C
_NP = 10112
_ROWS_PT = _NP // _NS
_NB = 3

_mesh = plsc.VectorSubcoreMesh(core_axis_name="c", subcore_axis_name="s")


@functools.partial(
    pl.kernel,
    out_type=jax.ShapeDtypeStruct((_NC, _NP, _D), jnp.float32),
    mesh=_mesh,
    compiler_params=pltpu.CompilerParams(use_tc_tiling_on_sc=False),
    scratch_types=[
        pltpu.VMEM((_NB, 2, _C), jnp.int32),
        pltpu.VMEM((_NB, _C), jnp.float32),
        pltpu.VMEM((_NB, _C, _D), jnp.float32),
        pltpu.VMEM_SHARED((_NP, _D), jnp.float32),
        [pltpu.SemaphoreType.DMA] * _NB,
        [pltpu.SemaphoreType.DMA] * _NB,
    ],
)
def _sc_gather_scatter(h_hbm, edata_hbm, kv_hbm, out_hbm,
                       ed_v, kv_v, rows_v, agg_sh, esems, gsems):
    cid = lax.axis_index("c")
    sid = lax.axis_index("s")
    base = (sid * _NC + cid) * _CPT

    def zbody(r, carry):
        for j in range(_D // 16):
            rows_v[0, r, pl.ds(j * 16, 16)] = jnp.zeros((16,), jnp.float32)
        return carry
    lax.fori_loop(0, _C, zbody, 0)
    row0 = sid * _ROWS_PT
    for j in range(5):
        pltpu.sync_copy(rows_v.at[0],
                        agg_sh.at[pl.ds(row0 + j * _C, _C)])
    pltpu.sync_copy(rows_v.at[0, pl.ds(0, 32)],
                    agg_sh.at[pl.ds(row0 + 5 * _C, 32)])
    plsc.subcore_barrier()

    def start_edata(k, b):
        kc = base + jnp.where(k < _CPT, k, 0)
        pltpu.async_copy(edata_hbm.at[kc], ed_v.at[b], esems[b])
        pltpu.async_copy(kv_hbm.at[kc], kv_v.at[b], esems[b])

    def wait_edata(b):
        pltpu.make_async_copy(edata_hbm.at[0], ed_v.at[b],
                              esems[b]).wait()
        pltpu.make_async_copy(kv_hbm.at[0], kv_v.at[b],
                              esems[b]).wait()

    def start_gather(b):
        pltpu.async_copy(h_hbm.at[ed_v.at[b, 0]], rows_v.at[b], gsems[b])

    def wait_gather(b):
        pltpu.make_async_copy(h_hbm.at[pl.ds(0, _C)], rows_v.at[b],
                              gsems[b]).wait()

    def scale_chunk(b):
        def gbody(g, carry):
            kv16 = kv_v[b, pl.ds(g * 16, 16)]
            for e16 in range(16):
                kv_e = kv16[e16]
                row = g * 16 + e16
                for j in range(_D // 16):
                    sl = pl.ds(j * 16, 16)
                    rows_v[b, row, sl] = rows_v[b, row, sl] * kv_e
            return carry
        lax.fori_loop(0, 7, gbody, 0)
        kv8 = kv_v[b, pl.ds(104, 16)]
        for e16 in range(8, 16):
            kv_e = kv8[e16]
            row = 104 + e16
            for j in range(_D // 16):
                sl = pl.ds(j * 16, 16)
                rows_v[b, row, sl] = rows_v[b, row, sl] * kv_e

    for b in range(_NB):
        start_edata(b, b)
    for b in range(_NB - 1):
        wait_edata(b)
        start_gather(b)

    def outer(i, carry):
        for u in range(_NB):
            k = _NB * i + u
            b = u
            bp = (u + _NB - 1) % _NB
            wait_edata(bp)
            start_gather(bp)
            wait_gather(b)
            scale_chunk(b)
            pltpu.sync_copy(rows_v.at[b], agg_sh.at[ed_v.at[b, 1]],
                            add=True)
            start_edata(k + _NB, b)
        return carry
    lax.fori_loop(0, _CPT // _NB, outer, 0)
    for j in range(_NB - 1):
        wait_gather((_CPT + j) % _NB)
    wait_edata((_CPT - 1) % _NB)

    plsc.subcore_barrier()
    pltpu.sync_copy(agg_sh.at[pl.ds(row0, _ROWS_PT)],
                    out_hbm.at[cid, pl.ds(row0, _ROWS_PT)])


_R = 1264


def _mm_tanh_body(q_ref, w_ref, o_ref):
    a = q_ref[0] + q_ref[1]
    o_ref[...] = jnp.tanh(jnp.dot(a, w_ref[...],
                                  preferred_element_type=jnp.float32))


def _tc_mm_tanh(q, w):
    return pl.pallas_call(
        _mm_tanh_body,
        grid=(_NP // _R,),
        in_specs=[
            pl.BlockSpec((2, _R, _D), lambda i: (0, i, 0)),
            pl.BlockSpec((_D, _D), lambda i: (0, 0)),
        ],
        out_specs=pl.BlockSpec((_R, _D), lambda i: (i, 0)),
        out_shape=jax.ShapeDtypeStruct((_NP, _D), jnp.float32),
    )(q, w)


def _final_body(x_ref, h1_ref, q_ref, w1_ref, we_ref, b_ref, o_ref):
    h2 = jnp.tanh(jnp.dot(q_ref[0] + q_ref[1], w1_ref[...],
                          preferred_element_type=jnp.float32))
    we = we_ref[...]
    acc = jnp.dot(x_ref[...], we[0:_D], preferred_element_type=jnp.float32)
    acc += jnp.dot(h1_ref[...], we[_D:2 * _D],
                   preferred_element_type=jnp.float32)
    acc += jnp.dot(h2, we[2 * _D:3 * _D], preferred_element_type=jnp.float32)
    o_ref[...] = acc + b_ref[...]


def _tc_final(x, h1, q, w1, w_enc, b_enc):
    return pl.pallas_call(
        _final_body,
        grid=(_NP // _R,),
        in_specs=[
            pl.BlockSpec((_R, _D), lambda i: (i, 0)),
            pl.BlockSpec((_R, _D), lambda i: (i, 0)),
            pl.BlockSpec((2, _R, _D), lambda i: (0, i, 0)),
            pl.BlockSpec((_D, _D), lambda i: (0, 0)),
            pl.BlockSpec((3 * _D, _D), lambda i: (0, 0)),
            pl.BlockSpec((1, _D), lambda i: (0, 0)),
        ],
        out_specs=pl.BlockSpec((_R, _D), lambda i: (i, 0)),
        out_shape=jax.ShapeDtypeStruct((_NP, _D), jnp.float32),
    )(x, h1, q, w1, w_enc, b_enc)


@jax.jit
def kernel(x, edge_index, kernel_vals, W0, W1, W_enc, b_enc):
    pad = _EPAD - _E
    src = jnp.concatenate([edge_index[0], jnp.zeros((pad,), jnp.int32)])
    dst = jnp.concatenate([edge_index[1], jnp.zeros((pad,), jnp.int32)])
    kv = jnp.concatenate([kernel_vals, jnp.zeros((pad,), jnp.float32)])
    edata = jnp.stack(
        [a.reshape(_TOTCH, _C) for a in (src, dst)], axis=1)
    kv_r = kv.reshape(_TOTCH, _C)
    x_p = jnp.pad(x, ((0, _NP - _N), (0, 0)))

    p = _sc_gather_scatter(x_p, edata, kv_r)
    h1 = _tc_mm_tanh(p, W0)
    q = _sc_gather_scatter(h1, edata, kv_r)
    emb = _tc_final(x_p, h1, q, W1, W_enc, b_enc.reshape(1, _D))
    return emb[:_N]
